# Initial kernel scaffold; baseline (speedup 1.0000x reference)
import jax
import jax.numpy as jnp
from jax.experimental import pallas as pl


def kernel(emb_table, edge_attrs, scale, edge_index):
    raise NotImplementedError("write your pallas kernel here")


# speedup vs baseline: 8.6110x; 8.6110x over previous
import functools

import jax
import jax.numpy as jnp
from jax import lax
from jax.experimental import pallas as pl
from jax.experimental.pallas import tpu as pltpu
from jax.experimental.pallas import tpu_sc as plsc

N_NODES = 50000
E = 800000
D = 64

HALF = 25000
PAD = 88
HPAD = HALF + PAD
NPAD = 2 * HPAD
TROWS = HPAD // 16
TRASH = HPAD
SROWS = HPAD + 8

CH = 128
KCH = 2
SUPE = CH * KCH
NSUP = E // SUPE
KMAX = -(-NSUP // 16)
NB = 56
NBLK = TROWS // NB

f32 = jnp.float32
i32 = jnp.int32


def _mesh():
    return plsc.VectorSubcoreMesh(core_axis_name="c", subcore_axis_name="s")


def _rsqrt16(x):
    i = lax.bitcast_convert_type(x, i32)
    i = jnp.int32(0x5F3759DF) - lax.shift_right_logical(i, 1)
    y = lax.bitcast_convert_type(i, f32)
    for _ in range(3):
        y = y * (jnp.float32(1.5) - jnp.float32(0.5) * x * y * y)
    return jnp.where(x > jnp.float32(0.0), y, jnp.float32(0.0))


def _fill_zeros(ref, rows, width):
    def body(r, _):
        for p in range(width // 16):
            ref[r, pl.ds(p * 16, 16)] = jnp.zeros((16,), f32)
        return 0
    lax.fori_loop(0, rows, body, 0)


def _dst_local(tbuf, didx2, lo):
    for j in range(KCH):
        for p in range(CH // 16):
            v = tbuf[pl.ds(j * CH + p * 16, 16)]
            inr = (v >= lo) & (v < lo + HALF)
            didx2[j, pl.ds(p * 16, 16)] = jnp.where(inr, v - lo, TRASH)


def _prep_body(to_hbm, emb_hbm, dis_out, y0_out,
               d_sh, tbuf, didx2, ones, zblk, dblk, disblk, embblk, y0blk,
               ssem):
    c = lax.axis_index("c")
    s = lax.axis_index("s")
    lo = c * HALF

    def fill_ones(r, _):
        ones[r] = jnp.ones((16,), f32)
        return 0
    lax.fori_loop(0, CH, fill_ones, 0)
    _fill_zeros(zblk, NB, 16)

    def zero_blk(j, _):
        pltpu.sync_copy(zblk, d_sh.at[pl.ds(s * TROWS + j * NB, NB)])
        return 0
    lax.fori_loop(0, NBLK, zero_blk, 0)
    plsc.subcore_barrier()

    def edge_iter(k, _):
        sup = s + 16 * k

        @pl.when(sup < NSUP)
        def _():
            base = sup * SUPE
            pltpu.sync_copy(to_hbm.at[pl.ds(base, SUPE)], tbuf)
            _dst_local(tbuf, didx2, lo)
            copies = [
                pltpu.async_copy(ones, d_sh.at[didx2.at[j]], ssem, add=True)
                for j in range(KCH)
            ]
            for cp in copies:
                cp.wait()
        return 0
    lax.fori_loop(0, KMAX, edge_iter, 0)
    plsc.subcore_barrier()

    def node_blk(j, _):
        lb = s * TROWS + j * NB
        pb = c * HPAD + lb
        pltpu.sync_copy(d_sh.at[pl.ds(lb, NB)], dblk)
        pltpu.sync_copy(emb_hbm.at[pl.ds(pb, NB)], embblk)

        def row(r, _):
            disv = _rsqrt16(dblk[r])
            disblk[r] = disv
            for p in range(D // 16):
                sl = pl.ds(p * 16, 16)
                y0blk[r, sl] = embblk[r, sl] * disv
            return 0
        lax.fori_loop(0, NB, row, 0)
        pltpu.sync_copy(disblk, dis_out.at[pl.ds(pb, NB)])
        pltpu.sync_copy(y0blk, y0_out.at[pl.ds(pb, NB)])
        return 0
    lax.fori_loop(0, NBLK, node_blk, 0)


def _layer_body(from_hbm, to_hbm, y_hbm, dis_hbm, acc_hbm, *refs, last):
    if last:
        (out_hbm, s_sh, fbuf, tbuf, didx2, rows, sblk, disblk,
         accblk, gsem, ssem) = refs
    else:
        (y_out, acc_out, s_sh, fbuf, tbuf, didx2, rows, sblk, disblk,
         accblk, gsem, ssem) = refs
    c = lax.axis_index("c")
    s = lax.axis_index("s")
    lo = c * HALF

    _fill_zeros(sblk, NB, D)

    def zero_blk(j, _):
        pltpu.sync_copy(sblk, s_sh.at[pl.ds(s * TROWS + j * NB, NB)])
        return 0
    lax.fori_loop(0, NBLK, zero_blk, 0)
    plsc.subcore_barrier()

    def edge_iter(k, _):
        sup = s + 16 * k

        @pl.when(sup < NSUP)
        def _():
            base = sup * SUPE
            pltpu.sync_copy(from_hbm.at[pl.ds(base, SUPE)], fbuf)
            pltpu.sync_copy(to_hbm.at[pl.ds(base, SUPE)], tbuf)
            _dst_local(tbuf, didx2, lo)
            for p in range(SUPE // 16):
                sl = pl.ds(p * 16, 16)
                f = fbuf[sl]
                fbuf[sl] = jnp.where(f >= HALF, f + PAD, f)
            gathers = [
                pltpu.async_copy(y_hbm.at[fbuf.at[pl.ds(j * CH, CH)]],
                                 rows.at[j], gsem)
                for j in range(KCH)
            ]
            for cp in gathers:
                cp.wait()
            scatters = [
                pltpu.async_copy(rows.at[j], s_sh.at[didx2.at[j]], ssem,
                                 add=True)
                for j in range(KCH)
            ]
            for cp in scatters:
                cp.wait()
        return 0
    lax.fori_loop(0, KMAX, edge_iter, 0)
    plsc.subcore_barrier()

    yblk = rows.at[0, pl.ds(0, NB)]

    def node_blk(j, _):
        lb = s * TROWS + j * NB
        pb = c * HPAD + lb
        pltpu.sync_copy(s_sh.at[pl.ds(lb, NB)], sblk)
        pltpu.sync_copy(dis_hbm.at[pl.ds(pb, NB)], disblk)
        pltpu.sync_copy(acc_hbm.at[pl.ds(pb, NB)], accblk)

        def row(r, _):
            dv = disblk[r]
            d2 = dv * dv
            for p in range(D // 16):
                sl = pl.ds(p * 16, 16)
                sv = sblk[r, sl]
                a = accblk[r, sl] + sv * dv
                if last:
                    accblk[r, sl] = a * jnp.float32(0.25)
                else:
                    accblk[r, sl] = a
                    yblk[r, sl] = sv * d2
            return 0
        lax.fori_loop(0, NB, row, 0)
        if last:
            pltpu.sync_copy(accblk, out_hbm.at[pl.ds(pb, NB)])
        else:
            pltpu.sync_copy(accblk, acc_out.at[pl.ds(pb, NB)])
            pltpu.sync_copy(yblk, y_out.at[pl.ds(pb, NB)])
        return 0
    lax.fori_loop(0, NBLK, node_blk, 0)


def _sds(shape, dtype=f32):
    return jax.ShapeDtypeStruct(shape, dtype)


def _make_prep():
    return pl.kernel(
        _prep_body,
        out_type=(_sds((NPAD, 16)), _sds((NPAD, D))),
        mesh=_mesh(),
        scratch_types=[
            pltpu.VMEM_SHARED((SROWS, 16), f32),
            pltpu.VMEM((SUPE,), i32),
            pltpu.VMEM((KCH, CH), i32),
            pltpu.VMEM((CH, 16), f32),
            pltpu.VMEM((NB, 16), f32),
            pltpu.VMEM((NB, 16), f32),
            pltpu.VMEM((NB, 16), f32),
            pltpu.VMEM((NB, D), f32),
            pltpu.VMEM((NB, D), f32),
            pltpu.SemaphoreType.DMA,
        ],
        compiler_params=pltpu.CompilerParams(use_tc_tiling_on_sc=False),
        name="lightgcn_prep",
    )


def _make_layer(last):
    if last:
        outs = _sds((NPAD, D))
    else:
        outs = (_sds((NPAD, D)), _sds((NPAD, D)))
    scratch = [
        pltpu.VMEM_SHARED((SROWS, D), f32),
        pltpu.VMEM((SUPE,), i32),
        pltpu.VMEM((SUPE,), i32),
        pltpu.VMEM((KCH, CH), i32),
        pltpu.VMEM((KCH, CH, D), f32),
        pltpu.VMEM((NB, D), f32),
        pltpu.VMEM((NB, 16), f32),
        pltpu.VMEM((NB, D), f32),
        pltpu.SemaphoreType.DMA,
        pltpu.SemaphoreType.DMA,
    ]
    return pl.kernel(
        functools.partial(_layer_body, last=last),
        out_type=outs,
        mesh=_mesh(),
        scratch_types=scratch,
        compiler_params=pltpu.CompilerParams(use_tc_tiling_on_sc=False),
        name="lightgcn_last" if last else "lightgcn_layer",
    )


def kernel(emb_table, edge_attrs, scale, edge_index):
    del edge_attrs, scale
    from_ = edge_index[0]
    to_ = edge_index[1]
    zpad = jnp.zeros((PAD, D), f32)
    emb_pad = jnp.concatenate(
        [emb_table[:HALF], zpad, emb_table[HALF:], zpad], axis=0)

    dis, y0 = _make_prep()(to_, emb_pad)
    mid = _make_layer(last=False)
    y1, a1 = mid(from_, to_, y0, dis, emb_pad)
    y2, a2 = mid(from_, to_, y1, dis, a1)
    outp = _make_layer(last=True)(from_, to_, y2, dis, a2)

    return jnp.concatenate([outp[:HALF], outp[HPAD:HPAD + HALF]], axis=0)
